# Initial kernel scaffold; baseline (speedup 1.0000x reference)
import jax
import jax.numpy as jnp
from jax.experimental import pallas as pl


def kernel(x, edge_index, W, b):
    raise NotImplementedError("write your pallas kernel here")


# speedup vs baseline: 5.4526x; 5.4526x over previous
import functools

import jax
import jax.numpy as jnp
from jax import lax
from jax.experimental import pallas as pl
from jax.experimental.pallas import tpu as pltpu
from jax.experimental.pallas import tpu_sc as plsc

N_NODES = 10000
N_EDGES = 320000
D = 128

NC = 2
NS = 16
NW = NC * NS

EDGES_PER_W = N_EDGES // NW
CHUNK = 80
NCHUNKS = EDGES_PER_W // CHUNK
ROWS_PER_SUB = 624
REM_ROWS = N_NODES - NS * ROWS_PER_SUB
REM_BASE = NS * ROWS_PER_SUB


def _sc_body(x_hbm, src_hbm, dst_hbm, zero_hbm, out_hbm,
             side_sh, src_v, dst_v, rows_v, gsem):
    c = lax.axis_index("c")
    s = lax.axis_index("s")
    wid = c * NS + s

    r0 = s * ROWS_PER_SUB
    pltpu.sync_copy(zero_hbm.at[pl.ds(r0, ROWS_PER_SUB)],
                    side_sh.at[pl.ds(r0, ROWS_PER_SUB)])

    @pl.when(s == 0)
    def _zero_rem():
        pltpu.sync_copy(zero_hbm.at[pl.ds(REM_BASE, REM_ROWS)],
                        side_sh.at[pl.ds(REM_BASE, REM_ROWS)])

    plsc.subcore_barrier()

    e_base = wid * EDGES_PER_W

    def chunk_step(i, carry):
        base = e_base + i * CHUNK
        pltpu.sync_copy(src_hbm.at[pl.ds(base, CHUNK)], src_v)
        pltpu.sync_copy(dst_hbm.at[pl.ds(base, CHUNK)], dst_v)
        pltpu.async_copy(x_hbm.at[src_v], rows_v, gsem).wait()
        pltpu.sync_copy(rows_v, side_sh.at[dst_v], add=True)
        return carry

    lax.fori_loop(0, NCHUNKS, chunk_step, 0)
    plsc.subcore_barrier()

    o0 = c * N_NODES + s * ROWS_PER_SUB
    pltpu.sync_copy(side_sh.at[pl.ds(r0, ROWS_PER_SUB)],
                    out_hbm.at[pl.ds(o0, ROWS_PER_SUB)])

    @pl.when(s == 0)
    def _out_rem():
        pltpu.sync_copy(side_sh.at[pl.ds(REM_BASE, REM_ROWS)],
                        out_hbm.at[pl.ds(c * N_NODES + REM_BASE, REM_ROWS)])


_sc_aggregate = functools.partial(
    pl.kernel,
    out_type=jax.ShapeDtypeStruct((NC * N_NODES, D), jnp.float32),
    mesh=plsc.VectorSubcoreMesh(core_axis_name="c", subcore_axis_name="s",
                                num_cores=NC, num_subcores=NS),
    scratch_types=[
        pltpu.VMEM_SHARED((N_NODES, D), jnp.float32),
        pltpu.VMEM((CHUNK,), jnp.int32),
        pltpu.VMEM((CHUNK,), jnp.int32),
        pltpu.VMEM((CHUNK, D), jnp.float32),
        pltpu.SemaphoreType.DMA,
    ],
)(_sc_body)


ROW_BLK = 1000


def _tc_body(x_ref, p0_ref, p1_ref, w_ref, b_ref, o_ref):
    emb = x_ref[...] + p0_ref[...] + p1_ref[...]
    h = jnp.dot(emb, w_ref[...], preferred_element_type=jnp.float32) + b_ref[...]
    o_ref[...] = jnp.where(h >= 0, h, 0.01 * h)


def _tc_finish(x, ps, W, b2):
    grid = (N_NODES // ROW_BLK,)
    return pl.pallas_call(
        _tc_body,
        grid=grid,
        in_specs=[
            pl.BlockSpec((ROW_BLK, D), lambda i: (i, 0)),
            pl.BlockSpec((ROW_BLK, D), lambda i: (i, 0)),
            pl.BlockSpec((ROW_BLK, D), lambda i: (i + N_NODES // ROW_BLK, 0)),
            pl.BlockSpec((D, D), lambda i: (0, 0)),
            pl.BlockSpec((1, D), lambda i: (0, 0)),
        ],
        out_specs=pl.BlockSpec((ROW_BLK, D), lambda i: (i, 0)),
        out_shape=jax.ShapeDtypeStruct((N_NODES, D), jnp.float32),
    )(x, ps, ps, W, b2)


def kernel(x, edge_index, W, b):
    ei = edge_index.astype(jnp.int32)
    src = ei[0]
    dst = ei[1]
    zeros = jnp.zeros((N_NODES, D), jnp.float32)
    ps = _sc_aggregate(x, src, dst, zeros)
    return _tc_finish(x, ps, W, b.reshape(1, D))


# speedup vs baseline: 12.7809x; 2.3440x over previous
import functools

import jax
import jax.numpy as jnp
from jax import lax
from jax.experimental import pallas as pl
from jax.experimental.pallas import tpu as pltpu
from jax.experimental.pallas import tpu_sc as plsc

N_NODES = 10000
N_EDGES = 320000
D = 128

NC = 2
NS = 16
NW = NC * NS

EDGES_PER_W = N_EDGES // NW
CHUNK = 128
NCHUNKS = EDGES_PER_W // CHUNK
TAIL = EDGES_PER_W - NCHUNKS * CHUNK
ROWS_PER_SUB = 624
REM_ROWS = N_NODES - NS * ROWS_PER_SUB
REM_BASE = NS * ROWS_PER_SUB


def _sc_body(x_hbm, src_hbm, dst_hbm, zero_hbm, out_hbm,
             side_sh, src_v, dst_v, dtail_v, rows_a, rows_b, gsem, isem):
    c = lax.axis_index("c")
    s = lax.axis_index("s")
    wid = c * NS + s
    e0 = wid * EDGES_PER_W

    r0 = s * ROWS_PER_SUB
    pltpu.sync_copy(zero_hbm.at[pl.ds(r0, ROWS_PER_SUB)],
                    side_sh.at[pl.ds(r0, ROWS_PER_SUB)])

    @pl.when(s == 0)
    def _zero_rem():
        pltpu.sync_copy(zero_hbm.at[pl.ds(REM_BASE, REM_ROWS)],
                        side_sh.at[pl.ds(REM_BASE, REM_ROWS)])

    pltpu.sync_copy(src_hbm.at[pl.ds(e0, EDGES_PER_W)], src_v)
    plsc.subcore_barrier()

    rows = (rows_a, rows_b)

    def issue(i, p):
        pltpu.async_copy(dst_hbm.at[pl.ds(e0 + i * CHUNK, CHUNK)],
                         dst_v.at[p], isem)
        pltpu.async_copy(x_hbm.at[src_v.at[pl.ds(i * CHUNK, CHUNK)]],
                         rows[p], gsem)

    def wait_pair(p):
        pltpu.make_async_copy(dst_hbm.at[pl.ds(0, CHUNK)],
                              dst_v.at[p], isem).wait()
        pltpu.make_async_copy(x_hbm.at[pl.ds(0, CHUNK)], rows[p], gsem).wait()

    issue(0, 0)
    issue(1, 1)

    @pl.loop(0, NCHUNKS // 2 - 1)
    def _outer(o):
        for p in (0, 1):
            i = 2 * o + p
            wait_pair(p)
            pltpu.sync_copy(rows[p], side_sh.at[dst_v.at[p]], add=True)
            issue(i + 2, p)

    for p in (0, 1):
        wait_pair(p)
        pltpu.sync_copy(rows[p], side_sh.at[dst_v.at[p]], add=True)

    pltpu.sync_copy(dst_hbm.at[pl.ds(e0 + NCHUNKS * CHUNK, TAIL)],
                    dtail_v.at[0])
    pltpu.async_copy(
        x_hbm.at[src_v.at[pl.ds(NCHUNKS * CHUNK, TAIL)]],
        rows_a.at[pl.ds(0, TAIL)], gsem).wait()
    pltpu.sync_copy(rows_a.at[pl.ds(0, TAIL)],
                    side_sh.at[dtail_v.at[0]], add=True)

    plsc.subcore_barrier()

    o0 = c * N_NODES + s * ROWS_PER_SUB
    pltpu.sync_copy(side_sh.at[pl.ds(r0, ROWS_PER_SUB)],
                    out_hbm.at[pl.ds(o0, ROWS_PER_SUB)])

    @pl.when(s == 0)
    def _out_rem():
        pltpu.sync_copy(side_sh.at[pl.ds(REM_BASE, REM_ROWS)],
                        out_hbm.at[pl.ds(c * N_NODES + REM_BASE, REM_ROWS)])


_sc_aggregate = functools.partial(
    pl.kernel,
    out_type=jax.ShapeDtypeStruct((NC * N_NODES, D), jnp.float32),
    mesh=plsc.VectorSubcoreMesh(core_axis_name="c", subcore_axis_name="s",
                                num_cores=NC, num_subcores=NS),
    scratch_types=[
        pltpu.VMEM_SHARED((N_NODES, D), jnp.float32),
        pltpu.VMEM((EDGES_PER_W,), jnp.int32),
        pltpu.VMEM((2, CHUNK), jnp.int32),
        pltpu.VMEM((1, TAIL), jnp.int32),
        pltpu.VMEM((CHUNK, D), jnp.float32),
        pltpu.VMEM((CHUNK, D), jnp.float32),
        pltpu.SemaphoreType.DMA,
        pltpu.SemaphoreType.DMA,
    ],
)(_sc_body)


ROW_BLK = 1000


def _tc_body(x_ref, p0_ref, p1_ref, w_ref, b_ref, o_ref):
    emb = x_ref[...] + p0_ref[...] + p1_ref[...]
    h = jnp.dot(emb, w_ref[...], preferred_element_type=jnp.float32) + b_ref[...]
    o_ref[...] = jnp.where(h >= 0, h, 0.01 * h)


def _tc_finish(x, ps, W, b2):
    grid = (N_NODES // ROW_BLK,)
    return pl.pallas_call(
        _tc_body,
        grid=grid,
        in_specs=[
            pl.BlockSpec((ROW_BLK, D), lambda i: (i, 0)),
            pl.BlockSpec((ROW_BLK, D), lambda i: (i, 0)),
            pl.BlockSpec((ROW_BLK, D), lambda i: (i + N_NODES // ROW_BLK, 0)),
            pl.BlockSpec((D, D), lambda i: (0, 0)),
            pl.BlockSpec((1, D), lambda i: (0, 0)),
        ],
        out_specs=pl.BlockSpec((ROW_BLK, D), lambda i: (i, 0)),
        out_shape=jax.ShapeDtypeStruct((N_NODES, D), jnp.float32),
    )(x, ps, ps, W, b2)


def kernel(x, edge_index, W, b):
    ei = edge_index.astype(jnp.int32)
    zeros = jnp.zeros((N_NODES, D), jnp.float32)
    ps = _sc_aggregate(x, ei[0], ei[1], zeros)
    return _tc_finish(x, ps, W, b.reshape(1, D))


# speedup vs baseline: 15.7723x; 1.2341x over previous
import functools

import jax
import jax.numpy as jnp
from jax import lax
from jax.experimental import pallas as pl
from jax.experimental.pallas import tpu as pltpu
from jax.experimental.pallas import tpu_sc as plsc

N_NODES = 10000
N_EDGES = 320000
D = 128

NC = 2
NS = 16
NW = NC * NS

EDGES_PER_W = N_EDGES // NW
CHUNK = 80
NCHUNKS = EDGES_PER_W // CHUNK
NBUF = 3
ROWS_PER_SUB = 624
REM_ROWS = N_NODES - NS * ROWS_PER_SUB
REM_BASE = NS * ROWS_PER_SUB
ZCOPIES = ROWS_PER_SUB // CHUNK
ZREM = ROWS_PER_SUB - ZCOPIES * CHUNK


def _sc_body(x_hbm, ei_hbm, out_hbm,
             side_sh, src_v, didx_v, rows_v, gsem, ssem, isem):
    c = lax.axis_index("c")
    s = lax.axis_index("s")
    wid = c * NS + s
    e0 = wid * EDGES_PER_W

    z16 = jnp.zeros((16,), jnp.float32)

    @pl.loop(0, CHUNK)
    def _zrow(r):
        for q in range(D // 16):
            rows_v[0, r, pl.ds(q * 16, 16)] = z16

    r0 = s * ROWS_PER_SUB
    for j in range(ZCOPIES):
        pltpu.async_copy(rows_v.at[0],
                         side_sh.at[pl.ds(r0 + j * CHUNK, CHUNK)], ssem)
    pltpu.async_copy(rows_v.at[0].at[pl.ds(0, ZREM)],
                     side_sh.at[pl.ds(r0 + ZCOPIES * CHUNK, ZREM)], ssem)

    @pl.when(s == 0)
    def _zero_rem():
        pltpu.sync_copy(rows_v.at[0].at[pl.ds(0, REM_ROWS)],
                        side_sh.at[pl.ds(REM_BASE, REM_ROWS)])

    pltpu.sync_copy(ei_hbm.at[pl.ds(e0, EDGES_PER_W)], src_v)
    for j in range(ZCOPIES):
        pltpu.make_async_copy(rows_v.at[0], side_sh.at[pl.ds(0, CHUNK)],
                              ssem).wait()
    pltpu.make_async_copy(rows_v.at[0].at[pl.ds(0, ZREM)],
                          side_sh.at[pl.ds(0, ZREM)], ssem).wait()
    plsc.subcore_barrier()

    d0 = N_EDGES + e0

    def issue(i, p):
        pltpu.async_copy(ei_hbm.at[pl.ds(d0 + i * CHUNK, CHUNK)],
                         didx_v.at[p], isem)
        pltpu.async_copy(x_hbm.at[src_v.at[pl.ds(i * CHUNK, CHUNK)]],
                         rows_v.at[p], gsem)

    def step(i, p, refill):
        pltpu.make_async_copy(x_hbm.at[pl.ds(0, CHUNK)], rows_v.at[p],
                              gsem).wait()
        pltpu.make_async_copy(ei_hbm.at[pl.ds(0, CHUNK)], didx_v.at[p],
                              isem).wait()
        pltpu.sync_copy(rows_v.at[p], side_sh.at[didx_v.at[p]], add=True)
        if refill:
            issue(i + NBUF, p)

    for p in range(NBUF):
        issue(p, p)

    @pl.loop(0, 40)
    def _outer(o):
        for p in range(NBUF):
            step(NBUF * o + p, p, True)

    step(120, 0, True)
    step(121, 1, True)
    step(122, 2, False)
    step(123, 0, False)
    step(124, 1, False)

    plsc.subcore_barrier()

    o0 = c * N_NODES + s * ROWS_PER_SUB
    pltpu.sync_copy(side_sh.at[pl.ds(r0, ROWS_PER_SUB)],
                    out_hbm.at[pl.ds(o0, ROWS_PER_SUB)])

    @pl.when(s == 0)
    def _out_rem():
        pltpu.sync_copy(side_sh.at[pl.ds(REM_BASE, REM_ROWS)],
                        out_hbm.at[pl.ds(c * N_NODES + REM_BASE, REM_ROWS)])


_sc_aggregate = functools.partial(
    pl.kernel,
    out_type=jax.ShapeDtypeStruct((NC * N_NODES, D), jnp.float32),
    mesh=plsc.VectorSubcoreMesh(core_axis_name="c", subcore_axis_name="s",
                                num_cores=NC, num_subcores=NS),
    scratch_types=[
        pltpu.VMEM_SHARED((N_NODES, D), jnp.float32),
        pltpu.VMEM((EDGES_PER_W,), jnp.int32),
        pltpu.VMEM((NBUF, CHUNK), jnp.int32),
        pltpu.VMEM((NBUF, CHUNK, D), jnp.float32),
        pltpu.SemaphoreType.DMA,
        pltpu.SemaphoreType.DMA,
        pltpu.SemaphoreType.DMA,
    ],
)(_sc_body)


ROW_BLK = 1000


def _tc_body(x_ref, p0_ref, p1_ref, w_ref, b_ref, o_ref):
    emb = x_ref[...] + p0_ref[...] + p1_ref[...]
    h = jnp.dot(emb, w_ref[...], preferred_element_type=jnp.float32) + b_ref[...]
    o_ref[...] = jnp.where(h >= 0, h, 0.01 * h)


def _tc_finish(x, ps, W, b2):
    grid = (N_NODES // ROW_BLK,)
    return pl.pallas_call(
        _tc_body,
        grid=grid,
        in_specs=[
            pl.BlockSpec((ROW_BLK, D), lambda i: (i, 0)),
            pl.BlockSpec((ROW_BLK, D), lambda i: (i, 0)),
            pl.BlockSpec((ROW_BLK, D), lambda i: (i + N_NODES // ROW_BLK, 0)),
            pl.BlockSpec((D, D), lambda i: (0, 0)),
            pl.BlockSpec((1, D), lambda i: (0, 0)),
        ],
        out_specs=pl.BlockSpec((ROW_BLK, D), lambda i: (i, 0)),
        out_shape=jax.ShapeDtypeStruct((N_NODES, D), jnp.float32),
    )(x, ps, ps, W, b2)


def kernel(x, edge_index, W, b):
    ei_flat = edge_index.astype(jnp.int32).reshape(-1)
    ps = _sc_aggregate(x, ei_flat)
    return _tc_finish(x, ps, W, b.reshape(1, D))
